# baseline (speedup 1.0000x reference)
import functools

import jax
import jax.numpy as jnp
from jax import lax
from jax.experimental import pallas as pl
from jax.experimental.pallas import tpu as pltpu
from jax.experimental.pallas import tpu_sc as plsc

_EMBED_DIM = 32
_FC_DIM = 200
_BATCH = 16384

_NC = 2
_NS = 16
_NW = _NC * _NS
_B_PER_W = _BATCH // _NW
_UVECS = _B_PER_W // 16
_ENTRIES = _B_PER_W * _EMBED_DIM
_CHUNK = 128
_NCHUNK = _ENTRIES // _CHUNK


def _make_gather():
    mesh = plsc.VectorSubcoreMesh(
        core_axis_name="c", subcore_axis_name="s",
        num_cores=_NC, num_subcores=_NS)

    @functools.partial(
        pl.kernel,
        mesh=mesh,
        out_type=jax.ShapeDtypeStruct((_EMBED_DIM, _BATCH), jnp.float32),
        scratch_types=[
            pltpu.VMEM((_B_PER_W,), jnp.int32),
            pltpu.VMEM((_EMBED_DIM, _B_PER_W), jnp.int32),
            pltpu.VMEM((_EMBED_DIM, _B_PER_W), jnp.float32),
            pltpu.SemaphoreType.DMA,
        ],
    )
    def gather(idx_hbm, table_hbm, out_hbm, idx_v, widx_v, col_v, sem):
        wid = lax.axis_index("s") * _NC + lax.axis_index("c")
        base = wid * _B_PER_W
        pltpu.sync_copy(idx_hbm.at[pl.ds(base, _B_PER_W)], idx_v)

        @pl.loop(0, _UVECS)
        def compute_indices(i):
            u16 = idx_v[pl.ds(i * 16, 16)]
            for r in range(_EMBED_DIM):
                widx_v[r, pl.ds(i * 16, 16)] = u16 + r * 1000000

        copies = []
        for j in range(_NCHUNK):
            r, c0 = divmod(j * _CHUNK, _B_PER_W)
            copies.append(pltpu.async_copy(
                table_hbm.at[widx_v.at[r, pl.ds(c0, _CHUNK)]],
                col_v.at[r, pl.ds(c0, _CHUNK)], sem))
        for c in copies:
            c.wait()
        pltpu.sync_copy(col_v, out_hbm.at[:, pl.ds(base, _B_PER_W)])

    return gather


_gather = _make_gather()


def _fc_body(wt_ref, embT_ref, b_ref, outT_ref):
    acc = jnp.dot(wt_ref[...], embT_ref[...],
                  preferred_element_type=jnp.float32)
    outT_ref[...] = jnp.maximum(acc + b_ref[...], 0.0)


def _fc(Wt, embT, b2d):
    blk = 2048
    return pl.pallas_call(
        _fc_body,
        grid=(_BATCH // blk,),
        in_specs=[
            pl.BlockSpec((_FC_DIM, _EMBED_DIM), lambda i: (0, 0)),
            pl.BlockSpec((_EMBED_DIM, blk), lambda i: (0, i)),
            pl.BlockSpec((_FC_DIM, 1), lambda i: (0, 0)),
        ],
        out_specs=pl.BlockSpec((_FC_DIM, blk), lambda i: (0, i)),
        out_shape=jax.ShapeDtypeStruct((_FC_DIM, _BATCH), jnp.float32),
    )(Wt, embT, b2d)


def kernel(indices, table, W, b):
    idx = indices.reshape(_BATCH).astype(jnp.int32)
    flat = table.T.reshape(32000000)
    embT = _gather(idx, flat)
    outT = _fc(W.T, embT, b.reshape(_FC_DIM, 1))
    return outT.T.reshape(_BATCH, 1, _FC_DIM)


# speedup vs baseline: 4.8214x; 4.8214x over previous
import functools

import jax
import jax.numpy as jnp
from jax import lax
from jax.experimental import pallas as pl
from jax.experimental.pallas import tpu as pltpu
from jax.experimental.pallas import tpu_sc as plsc

_EMBED_DIM = 32
_FC_DIM = 200
_BATCH = 16384

_NC = 2
_NS = 16
_NW = _NC * _NS
_B_PER_W = _BATCH // _NW
_UVECS = _B_PER_W // 16
_ENTRIES = _B_PER_W * _EMBED_DIM
_CHUNK = 128
_NCHUNK = _ENTRIES // _CHUNK


def _make_gather():
    mesh = plsc.VectorSubcoreMesh(
        core_axis_name="c", subcore_axis_name="s",
        num_cores=_NC, num_subcores=_NS)

    @functools.partial(
        pl.kernel,
        mesh=mesh,
        out_type=jax.ShapeDtypeStruct((_EMBED_DIM, _BATCH), jnp.float32),
        scratch_types=[
            pltpu.VMEM((_B_PER_W,), jnp.int32),
            pltpu.VMEM((_B_PER_W,), jnp.int32),
            pltpu.VMEM((_B_PER_W, 128), jnp.float32),
            pltpu.VMEM((_EMBED_DIM, _B_PER_W), jnp.float32),
            pltpu.SemaphoreType.DMA,
        ],
        compiler_params=pltpu.CompilerParams(needs_layout_passes=False),
    )
    def gather(idx_hbm, table_hbm, out_hbm, idx_v, qidx_v, rows_v, col_v,
               sem):
        wid = lax.axis_index("s") * _NC + lax.axis_index("c")
        base = wid * _B_PER_W
        pltpu.sync_copy(idx_hbm.at[pl.ds(base, _B_PER_W)], idx_v)

        two = jnp.full((16,), 2, jnp.int32)

        @pl.loop(0, _UVECS)
        def compute_rows(i):
            u16 = idx_v[pl.ds(i * 16, 16)]
            qidx_v[pl.ds(i * 16, 16)] = lax.shift_right_logical(u16, two)

        copies = []
        for j in range(_B_PER_W // _CHUNK):
            copies.append(pltpu.async_copy(
                table_hbm.at[qidx_v.at[pl.ds(j * _CHUNK, _CHUNK)]],
                rows_v.at[pl.ds(j * _CHUNK, _CHUNK)], sem))
        for c in copies:
            c.wait()

        lanes = lax.iota(jnp.int32, 16)
        three = jnp.full((16,), 3, jnp.int32)

        @pl.loop(0, _UVECS)
        def extract(i):
            u16 = idx_v[pl.ds(i * 16, 16)]
            rowv = i * 16 + lanes
            colbase = jnp.bitwise_and(u16, three) * 32
            for r in range(_EMBED_DIM):
                vals = plsc.load_gather(rows_v, [rowv, colbase + r])
                col_v[r, pl.ds(i * 16, 16)] = vals

        pltpu.sync_copy(col_v, out_hbm.at[:, pl.ds(base, _B_PER_W)])

    return gather


_gather = _make_gather()


def _fc_body(wt_ref, embT_ref, b_ref, outT_ref):
    acc = jnp.dot(wt_ref[...], embT_ref[...],
                  preferred_element_type=jnp.float32)
    outT_ref[...] = jnp.maximum(acc + b_ref[...], 0.0)


def _fc(Wt, embT, b2d):
    blk = 2048
    return pl.pallas_call(
        _fc_body,
        grid=(_BATCH // blk,),
        in_specs=[
            pl.BlockSpec((_FC_DIM, _EMBED_DIM), lambda i: (0, 0)),
            pl.BlockSpec((_EMBED_DIM, blk), lambda i: (0, i)),
            pl.BlockSpec((_FC_DIM, 1), lambda i: (0, 0)),
        ],
        out_specs=pl.BlockSpec((_FC_DIM, blk), lambda i: (0, i)),
        out_shape=jax.ShapeDtypeStruct((_FC_DIM, _BATCH), jnp.float32),
    )(Wt, embT, b2d)


def kernel(indices, table, W, b):
    idx = indices.reshape(_BATCH).astype(jnp.int32)
    flat = table.reshape(250000, 128)
    embT = _gather(idx, flat)
    outT = _fc(W.T, embT, b.reshape(_FC_DIM, 1))
    return outT.T.reshape(_BATCH, 1, _FC_DIM)


# speedup vs baseline: 18.1453x; 3.7635x over previous
import functools

import jax
import jax.numpy as jnp
from jax import lax
from jax.experimental import pallas as pl
from jax.experimental.pallas import tpu as pltpu
from jax.experimental.pallas import tpu_sc as plsc

_EMBED_DIM = 32
_FC_DIM = 200
_BATCH = 16384

_NC = 2
_NS = 16
_NW = _NC * _NS
_B_PER_W = _BATCH // _NW
_UVECS = _B_PER_W // 16


def _make_gather():
    mesh = plsc.VectorSubcoreMesh(
        core_axis_name="c", subcore_axis_name="s",
        num_cores=_NC, num_subcores=_NS)

    @functools.partial(
        pl.kernel,
        mesh=mesh,
        out_type=jax.ShapeDtypeStruct((_EMBED_DIM, _BATCH), jnp.float32),
        scratch_types=[
            pltpu.VMEM((_B_PER_W,), jnp.int32),
            pltpu.VMEM((_EMBED_DIM, 16 * 128), jnp.float32),
            pltpu.VMEM((_EMBED_DIM, _B_PER_W), jnp.float32),
            pltpu.SemaphoreType.DMA,
        ],
        compiler_params=pltpu.CompilerParams(
            needs_layout_passes=False,
            use_tc_tiling_on_sc=True),
    )
    def gather(idx_hbm, tableT_hbm, out_hbm, idx_v, stage_v, col_v, sem):
        wid = lax.axis_index("s") * _NC + lax.axis_index("c")
        base = wid * _B_PER_W
        pltpu.sync_copy(idx_hbm.at[pl.ds(base, _B_PER_W)], idx_v)

        lanes = lax.iota(jnp.int32, 16)
        l127 = jnp.full((16,), 127, jnp.int32)

        @pl.loop(0, _UVECS)
        def group(i):
            vec16 = idx_v[pl.ds(i * 16, 16)]
            colv = lanes * 128 + jnp.bitwise_and(vec16, l127)
            copies = []
            for l in range(16):
                u = vec16[l]
                cb = pl.multiple_of(u - lax.bitwise_and(u, 127), 128)
                copies.append(pltpu.async_copy(
                    tableT_hbm.at[:, pl.ds(cb, 128)],
                    stage_v.at[:, pl.ds(l * 128, 128)], sem))
            for c in copies:
                c.wait()
            for r in range(_EMBED_DIM):
                rv = jnp.full((16,), r, jnp.int32)
                vals = plsc.load_gather(stage_v, [rv, colv])
                col_v[r, pl.ds(i * 16, 16)] = vals

        pltpu.sync_copy(col_v, out_hbm.at[:, pl.ds(base, _B_PER_W)])

    return gather


_gather = _make_gather()


def _fc_body(wt_ref, embT_ref, b_ref, outT_ref):
    acc = jnp.dot(wt_ref[...], embT_ref[...],
                  preferred_element_type=jnp.float32)
    outT_ref[...] = jnp.maximum(acc + b_ref[...], 0.0)


def _fc(Wt, embT, b2d):
    blk = 2048
    return pl.pallas_call(
        _fc_body,
        grid=(_BATCH // blk,),
        in_specs=[
            pl.BlockSpec((_FC_DIM, _EMBED_DIM), lambda i: (0, 0)),
            pl.BlockSpec((_EMBED_DIM, blk), lambda i: (0, i)),
            pl.BlockSpec((_FC_DIM, 1), lambda i: (0, 0)),
        ],
        out_specs=pl.BlockSpec((_FC_DIM, blk), lambda i: (0, i)),
        out_shape=jax.ShapeDtypeStruct((_FC_DIM, _BATCH), jnp.float32),
    )(Wt, embT, b2d)


def kernel(indices, table, W, b):
    idx = indices.reshape(_BATCH).astype(jnp.int32)
    embT = _gather(idx, table.T)
    outT = _fc(W.T, embT, b.reshape(_FC_DIM, 1))
    return outT.T.reshape(_BATCH, 1, _FC_DIM)


# speedup vs baseline: 18.9424x; 1.0439x over previous
import functools

import jax
import jax.numpy as jnp
from jax import lax
from jax.experimental import pallas as pl
from jax.experimental.pallas import tpu as pltpu
from jax.experimental.pallas import tpu_sc as plsc

_EMBED_DIM = 32
_FC_DIM = 200
_BATCH = 16384

_NC = 2
_NS = 16
_NW = _NC * _NS
_B_PER_W = _BATCH // _NW
_UVECS = _B_PER_W // 16


def _make_gather():
    mesh = plsc.VectorSubcoreMesh(
        core_axis_name="c", subcore_axis_name="s",
        num_cores=_NC, num_subcores=_NS)

    @functools.partial(
        pl.kernel,
        mesh=mesh,
        out_type=jax.ShapeDtypeStruct((_EMBED_DIM, _BATCH), jnp.float32),
        scratch_types=[
            pltpu.VMEM((_B_PER_W,), jnp.int32),
            pltpu.VMEM((_EMBED_DIM, 16 * 128), jnp.float32),
            pltpu.VMEM((_EMBED_DIM, _B_PER_W), jnp.float32),
            pltpu.SemaphoreType.DMA((16,)),
        ],
        compiler_params=pltpu.CompilerParams(
            needs_layout_passes=False,
            use_tc_tiling_on_sc=True),
    )
    def gather(idx_hbm, tableT_hbm, out_hbm, idx_v, stage_v, col_v, sems):
        wid = lax.axis_index("s") * _NC + lax.axis_index("c")
        base = wid * _B_PER_W
        pltpu.sync_copy(idx_hbm.at[pl.ds(base, _B_PER_W)], idx_v)

        lanes = lax.iota(jnp.int32, 16)

        def issue(u, l):
            cb = pl.multiple_of(u - lax.bitwise_and(u, 127), 128)
            pltpu.async_copy(
                tableT_hbm.at[:, pl.ds(cb, 128)],
                stage_v.at[:, pl.ds(l * 128, 128)], sems.at[l])

        vec0 = idx_v[pl.ds(0, 16)]
        for l in range(16):
            issue(vec0[l], l)

        @pl.loop(0, _UVECS)
        def group(i):
            vec = idx_v[pl.ds(i * 16, 16)]
            nxt = jnp.minimum(i + 1, _UVECS - 1)
            vec_n = idx_v[pl.ds(nxt * 16, 16)]
            for l in range(16):
                pltpu.make_async_copy(
                    tableT_hbm.at[:, pl.ds(0, 128)],
                    stage_v.at[:, pl.ds(l * 128, 128)], sems.at[l]).wait()
                u = vec[l]
                colb = jnp.broadcast_to(
                    l * 128 + lax.bitwise_and(u, 127), (16,))
                v1 = plsc.load_gather(stage_v, [lanes, colb])
                v2 = plsc.load_gather(stage_v, [lanes + 16, colb])
                cpos = jnp.broadcast_to(i * 16 + l, (16,))
                plsc.store_scatter(col_v, [lanes, cpos], v1)
                plsc.store_scatter(col_v, [lanes + 16, cpos], v2)

                @pl.when(i < _UVECS - 1)
                def _():
                    issue(vec_n[l], l)

        pltpu.sync_copy(col_v, out_hbm.at[:, pl.ds(base, _B_PER_W)])

    return gather


_gather = _make_gather()


def _fc_body(wt_ref, embT_ref, b_ref, outT_ref):
    acc = jnp.dot(wt_ref[...], embT_ref[...],
                  preferred_element_type=jnp.float32)
    outT_ref[...] = jnp.maximum(acc + b_ref[...], 0.0)


def _fc(Wt, embT, b2d):
    blk = 2048
    return pl.pallas_call(
        _fc_body,
        grid=(_BATCH // blk,),
        in_specs=[
            pl.BlockSpec((_FC_DIM, _EMBED_DIM), lambda i: (0, 0)),
            pl.BlockSpec((_EMBED_DIM, blk), lambda i: (0, i)),
            pl.BlockSpec((_FC_DIM, 1), lambda i: (0, 0)),
        ],
        out_specs=pl.BlockSpec((_FC_DIM, blk), lambda i: (0, i)),
        out_shape=jax.ShapeDtypeStruct((_FC_DIM, _BATCH), jnp.float32),
    )(Wt, embT, b2d)


def kernel(indices, table, W, b):
    idx = indices.reshape(_BATCH).astype(jnp.int32)
    embT = _gather(idx, table.T)
    outT = _fc(W.T, embT, b.reshape(_FC_DIM, 1))
    return outT.T.reshape(_BATCH, 1, _FC_DIM)


# speedup vs baseline: 18.9537x; 1.0006x over previous
import functools

import jax
import jax.numpy as jnp
from jax import lax
from jax.experimental import pallas as pl
from jax.experimental.pallas import tpu as pltpu
from jax.experimental.pallas import tpu_sc as plsc

_EMBED_DIM = 32
_FC_DIM = 200
_BATCH = 16384

_NC = 2
_NS = 16
_NW = _NC * _NS
_B_PER_W = _BATCH // _NW
_UVECS = _B_PER_W // 16


def _make_gather():
    mesh = plsc.VectorSubcoreMesh(
        core_axis_name="c", subcore_axis_name="s",
        num_cores=_NC, num_subcores=_NS)

    @functools.partial(
        pl.kernel,
        mesh=mesh,
        out_type=jax.ShapeDtypeStruct((_EMBED_DIM, _BATCH), jnp.float32),
        scratch_types=[
            pltpu.VMEM((_B_PER_W,), jnp.int32),
            pltpu.VMEM((_EMBED_DIM, 16 * 128), jnp.float32),
            pltpu.VMEM((_EMBED_DIM, _B_PER_W), jnp.float32),
            pltpu.SemaphoreType.DMA((16,)),
        ],
        compiler_params=pltpu.CompilerParams(
            needs_layout_passes=False,
            use_tc_tiling_on_sc=True),
    )
    def gather(idx_hbm, tableT_hbm, out_hbm, idx_v, stage_v, col_v, sems):
        wid = lax.axis_index("s") * _NC + lax.axis_index("c")
        base = wid * _B_PER_W
        pltpu.sync_copy(idx_hbm.at[pl.ds(base, _B_PER_W)], idx_v)

        lanes = lax.iota(jnp.int32, 16)

        def issue(u, l):
            cb = pl.multiple_of(u - lax.bitwise_and(u, 127), 128)
            for g in range(4):
                pltpu.async_copy(
                    tableT_hbm.at[pl.ds(g * 8, 8), pl.ds(cb, 128)],
                    stage_v.at[pl.ds(g * 8, 8), pl.ds(l * 128, 128)],
                    sems.at[l])

        vec0 = idx_v[pl.ds(0, 16)]
        for l in range(16):
            issue(vec0[l], l)

        @pl.loop(0, _UVECS)
        def group(i):
            vec = idx_v[pl.ds(i * 16, 16)]
            nxt = jnp.minimum(i + 1, _UVECS - 1)
            vec_n = idx_v[pl.ds(nxt * 16, 16)]
            for l in range(16):
                pltpu.make_async_copy(
                    tableT_hbm.at[:, pl.ds(0, 128)],
                    stage_v.at[:, pl.ds(l * 128, 128)], sems.at[l]).wait()
                u = vec[l]
                colb = jnp.broadcast_to(
                    l * 128 + lax.bitwise_and(u, 127), (16,))
                v1 = plsc.load_gather(stage_v, [lanes, colb])
                v2 = plsc.load_gather(stage_v, [lanes + 16, colb])
                cpos = jnp.broadcast_to(i * 16 + l, (16,))
                plsc.store_scatter(col_v, [lanes, cpos], v1)
                plsc.store_scatter(col_v, [lanes + 16, cpos], v2)

                @pl.when(i < _UVECS - 1)
                def _():
                    issue(vec_n[l], l)

        pltpu.sync_copy(col_v, out_hbm.at[:, pl.ds(base, _B_PER_W)])

    return gather


_gather = _make_gather()


def _fc_body(wt_ref, embT_ref, b_ref, outT_ref):
    acc = jnp.dot(wt_ref[...], embT_ref[...],
                  preferred_element_type=jnp.float32)
    outT_ref[...] = jnp.maximum(acc + b_ref[...], 0.0)


def _fc(Wt, embT, b2d):
    blk = 2048
    return pl.pallas_call(
        _fc_body,
        grid=(_BATCH // blk,),
        in_specs=[
            pl.BlockSpec((_FC_DIM, _EMBED_DIM), lambda i: (0, 0)),
            pl.BlockSpec((_EMBED_DIM, blk), lambda i: (0, i)),
            pl.BlockSpec((_FC_DIM, 1), lambda i: (0, 0)),
        ],
        out_specs=pl.BlockSpec((_FC_DIM, blk), lambda i: (0, i)),
        out_shape=jax.ShapeDtypeStruct((_FC_DIM, _BATCH), jnp.float32),
    )(Wt, embT, b2d)


def kernel(indices, table, W, b):
    idx = indices.reshape(_BATCH).astype(jnp.int32)
    embT = _gather(idx, table.T)
    outT = _fc(W.T, embT, b.reshape(_FC_DIM, 1))
    return outT.T.reshape(_BATCH, 1, _FC_DIM)


# speedup vs baseline: 19.2114x; 1.0136x over previous
import functools

import jax
import jax.numpy as jnp
from jax import lax
from jax.experimental import pallas as pl
from jax.experimental.pallas import tpu as pltpu
from jax.experimental.pallas import tpu_sc as plsc

_EMBED_DIM = 32
_FC_DIM = 200
_BATCH = 16384

_NC = 2
_NS = 16
_NW = _NC * _NS
_B_PER_W = _BATCH // _NW
_UVECS = _B_PER_W // 16


def _make_gather():
    mesh = plsc.VectorSubcoreMesh(
        core_axis_name="c", subcore_axis_name="s",
        num_cores=_NC, num_subcores=_NS)

    @functools.partial(
        pl.kernel,
        mesh=mesh,
        out_type=jax.ShapeDtypeStruct((_EMBED_DIM, _BATCH), jnp.float32),
        scratch_types=[
            pltpu.VMEM((_B_PER_W,), jnp.int32),
            pltpu.VMEM((_EMBED_DIM, 16 * 128), jnp.float32),
            pltpu.VMEM((_EMBED_DIM, _B_PER_W), jnp.float32),
            pltpu.SemaphoreType.DMA((16,)),
        ],
        compiler_params=pltpu.CompilerParams(
            needs_layout_passes=False,
            use_tc_tiling_on_sc=True),
    )
    def gather(idx_hbm, tableT_hbm, out_hbm, idx_v, stage_v, col_v, sems):
        wid = lax.axis_index("s") * _NC + lax.axis_index("c")
        base = wid * _B_PER_W
        pltpu.sync_copy(idx_hbm.at[pl.ds(base, _B_PER_W)], idx_v)

        lanes = lax.iota(jnp.int32, 16)

        def issue(u, l):
            cb = pl.multiple_of(u - lax.bitwise_and(u, 127), 128)
            for g in range(4):
                pltpu.async_copy(
                    tableT_hbm.at[pl.ds(g * 8, 8), pl.ds(cb, 128)],
                    stage_v.at[pl.ds(g * 8, 8), pl.ds(l * 128, 128)],
                    sems.at[l])

        vec0 = idx_v[pl.ds(0, 16)]
        for l in range(16):
            issue(vec0[l], l)

        @pl.loop(0, _UVECS)
        def group(i):
            vec = idx_v[pl.ds(i * 16, 16)]
            nxt = jnp.minimum(i + 1, _UVECS - 1)
            vec_n = idx_v[pl.ds(nxt * 16, 16)]
            for l in range(16):
                pltpu.make_async_copy(
                    tableT_hbm.at[:, pl.ds(0, 128)],
                    stage_v.at[:, pl.ds(l * 128, 128)], sems.at[l]).wait()
                u = vec[l]
                colb = jnp.broadcast_to(
                    l * 128 + lax.bitwise_and(u, 127), (16,))
                v1 = plsc.load_gather(stage_v, [lanes, colb])
                v2 = plsc.load_gather(stage_v, [lanes + 16, colb])
                cpos = jnp.broadcast_to(i * 16 + l, (16,))
                plsc.store_scatter(col_v, [lanes, cpos], v1)
                plsc.store_scatter(col_v, [lanes + 16, cpos], v2)

                @pl.when(i < _UVECS - 1)
                def _():
                    issue(vec_n[l], l)

        pltpu.sync_copy(col_v, out_hbm.at[:, pl.ds(base, _B_PER_W)])

    return gather


_gather = _make_gather()


def _fc_body(wt_ref, embT_ref, b_ref, outT_ref):
    acc = jnp.dot(wt_ref[...], embT_ref[...],
                  preferred_element_type=jnp.float32)
    outT_ref[...] = jnp.maximum(acc + b_ref[...], 0.0)


def _fc(Wt, embT, b2d):
    blk = 4096
    return pl.pallas_call(
        _fc_body,
        grid=(_BATCH // blk,),
        in_specs=[
            pl.BlockSpec((_FC_DIM, _EMBED_DIM), lambda i: (0, 0)),
            pl.BlockSpec((_EMBED_DIM, blk), lambda i: (0, i)),
            pl.BlockSpec((_FC_DIM, 1), lambda i: (0, 0)),
        ],
        out_specs=pl.BlockSpec((_FC_DIM, blk), lambda i: (0, i)),
        out_shape=jax.ShapeDtypeStruct((_FC_DIM, _BATCH), jnp.float32),
    )(Wt, embT, b2d)


def kernel(indices, table, W, b):
    idx = indices.reshape(_BATCH).astype(jnp.int32)
    embT = _gather(idx, table.T)
    outT = _fc(W.T, embT, b.reshape(_FC_DIM, 1))
    return outT.T.reshape(_BATCH, 1, _FC_DIM)


# speedup vs baseline: 19.2566x; 1.0024x over previous
import functools

import jax
import jax.numpy as jnp
from jax import lax
from jax.experimental import pallas as pl
from jax.experimental.pallas import tpu as pltpu
from jax.experimental.pallas import tpu_sc as plsc

_EMBED_DIM = 32
_FC_DIM = 200
_BATCH = 16384

_NC = 2
_NS = 16
_NW = _NC * _NS
_B_PER_W = _BATCH // _NW
_UVECS = _B_PER_W // 16


def _make_gather():
    mesh = plsc.VectorSubcoreMesh(
        core_axis_name="c", subcore_axis_name="s",
        num_cores=_NC, num_subcores=_NS)

    @functools.partial(
        pl.kernel,
        mesh=mesh,
        out_type=jax.ShapeDtypeStruct((_EMBED_DIM, _BATCH), jnp.float32),
        scratch_types=[
            pltpu.VMEM((_B_PER_W,), jnp.int32),
            pltpu.VMEM((_EMBED_DIM, 16 * 128), jnp.float32),
            pltpu.VMEM((_EMBED_DIM, _B_PER_W), jnp.float32),
            pltpu.SemaphoreType.DMA((16,)),
        ],
        compiler_params=pltpu.CompilerParams(
            needs_layout_passes=False,
            use_tc_tiling_on_sc=True),
    )
    def gather(idx_hbm, tableT_hbm, out_hbm, idx_v, stage_v, col_v, sems):
        wid = lax.axis_index("s") * _NC + lax.axis_index("c")
        base = wid * _B_PER_W
        pltpu.sync_copy(idx_hbm.at[pl.ds(base, _B_PER_W)], idx_v)

        lanes = lax.iota(jnp.int32, 16)

        def issue(u, l):
            cb = pl.multiple_of(u - lax.bitwise_and(u, 127), 128)
            for g in range(4):
                pltpu.async_copy(
                    tableT_hbm.at[pl.ds(g * 8, 8), pl.ds(cb, 128)],
                    stage_v.at[pl.ds(g * 8, 8), pl.ds(l * 128, 128)],
                    sems.at[l])

        vec0 = idx_v[pl.ds(0, 16)]
        for l in range(16):
            issue(vec0[l], l)

        @pl.loop(0, _UVECS)
        def group(i):
            vec = idx_v[pl.ds(i * 16, 16)]
            nxt = jnp.minimum(i + 1, _UVECS - 1)
            vec_n = idx_v[pl.ds(nxt * 16, 16)]
            for l in range(16):
                pltpu.make_async_copy(
                    tableT_hbm.at[:, pl.ds(0, 128)],
                    stage_v.at[:, pl.ds(l * 128, 128)], sems.at[l]).wait()
                u = vec[l]
                colb = jnp.broadcast_to(
                    l * 128 + lax.bitwise_and(u, 127), (16,))
                v1 = plsc.load_gather(stage_v, [lanes, colb])
                v2 = plsc.load_gather(stage_v, [lanes + 16, colb])
                cpos = jnp.broadcast_to(i * 16 + l, (16,))
                plsc.store_scatter(col_v, [lanes, cpos], v1)
                plsc.store_scatter(col_v, [lanes + 16, cpos], v2)

                @pl.when(i < _UVECS - 1)
                def _():
                    issue(vec_n[l], l)

        pltpu.sync_copy(col_v, out_hbm.at[:, pl.ds(base, _B_PER_W)])

    return gather


_gather = _make_gather()


def _fc_body(wt_ref, embT_ref, b_ref, outT_ref):
    acc = jnp.dot(wt_ref[...], embT_ref[...],
                  preferred_element_type=jnp.float32)
    outT_ref[...] = jnp.maximum(acc + b_ref[...], 0.0)


def _fc(Wt, embT, b2d):
    blk = 8192
    return pl.pallas_call(
        _fc_body,
        grid=(_BATCH // blk,),
        in_specs=[
            pl.BlockSpec((_FC_DIM, _EMBED_DIM), lambda i: (0, 0)),
            pl.BlockSpec((_EMBED_DIM, blk), lambda i: (0, i)),
            pl.BlockSpec((_FC_DIM, 1), lambda i: (0, 0)),
        ],
        out_specs=pl.BlockSpec((_FC_DIM, blk), lambda i: (0, i)),
        out_shape=jax.ShapeDtypeStruct((_FC_DIM, _BATCH), jnp.float32),
    )(Wt, embT, b2d)


def kernel(indices, table, W, b):
    idx = indices.reshape(_BATCH).astype(jnp.int32)
    embT = _gather(idx, table.T)
    outT = _fc(W.T, embT, b.reshape(_FC_DIM, 1))
    return outT.T.reshape(_BATCH, 1, _FC_DIM)
